# Initial kernel scaffold; baseline (speedup 1.0000x reference)
import jax
import jax.numpy as jnp
from jax.experimental import pallas as pl


def kernel(Corpus_, entity_embeddings, relation_embed, edge_list, edge_type, edge_embed, edge_list_nhop, edge_type_nhop, a_heads, a2_heads, W, a_out, a2_out):
    raise NotImplementedError("write your pallas kernel here")


# speedup vs baseline: 1.4214x; 1.4214x over previous
import functools

import jax
import jax.numpy as jnp
from jax import lax
from jax.experimental import pallas as pl
from jax.experimental.pallas import tpu as pltpu
from jax.experimental.pallas import tpu_sc as plsc

NHEADS = 2
ALPHA = 0.2
N_NODES = 10000
NFEAT = 128
NHID = 64
REL_DIM = 128
N_REL = 200
E1 = 160000
E2 = 32000

HC = 64
HROW = 80
SSW = 16
NW = 32
C = 128
E1P = 163840
E2P = 32768
TP = 208
NP = 10240
RPS = NP // 16


def _mm_body(x_ref, b_ref, o_ref):
    o_ref[...] = jnp.dot(x_ref[...], b_ref[...],
                         preferred_element_type=jnp.float32)


def _mm(x, b, bm):
    m, k = x.shape
    n = b.shape[1]
    mp = ((m + bm - 1) // bm) * bm
    if mp != m:
        x = jnp.pad(x, ((0, mp - m), (0, 0)))
    out = pl.pallas_call(
        _mm_body,
        grid=(mp // bm,),
        in_specs=[pl.BlockSpec((bm, k), lambda i: (i, 0)),
                  pl.BlockSpec((k, n), lambda i: (0, 0))],
        out_specs=pl.BlockSpec((bm, n), lambda i: (i, 0)),
        out_shape=jax.ShapeDtypeStruct((mp, n), jnp.float32),
    )(x, b)
    return out[:m] if mp != m else out


def _combine1_body(acc_ref, xs_ref, b4_ref, p2_ref):
    acc = acc_ref[...]
    num = jnp.concatenate([acc[0, :, :HC], acc[1, :, :HC]], axis=1)
    w = jnp.concatenate([acc[0, :, HC:HC + 1], acc[1, :, HC:HC + 1]], axis=1)
    rs = jnp.where(w == 0.0, 1e-12, w)
    bm = num.shape[0]
    wf = jnp.concatenate(
        [jnp.broadcast_to(w[:, h:h + 1], (bm, NHID)) for h in range(NHEADS)],
        axis=1)
    rsf = jnp.concatenate(
        [jnp.broadcast_to(rs[:, h:h + 1], (bm, NHID)) for h in range(NHEADS)],
        axis=1)
    h = (wf * xs_ref[...] + num) / rsf
    x2 = jnp.where(h > 0, h, jnp.exp(jnp.minimum(h, 0.0)) - 1.0)
    p2_ref[...] = jnp.dot(x2, b4_ref[...], preferred_element_type=jnp.float32)


def _combine1(acc, xs, b4, bm=512):
    n = b4.shape[1]
    xs = jnp.pad(xs, ((0, NP - N_NODES), (0, 0)))
    out = pl.pallas_call(
        _combine1_body,
        grid=(NP // bm,),
        in_specs=[pl.BlockSpec((2, bm, HROW), lambda i: (0, i, 0)),
                  pl.BlockSpec((bm, 128), lambda i: (i, 0)),
                  pl.BlockSpec((128, n), lambda i: (0, 0))],
        out_specs=pl.BlockSpec((bm, n), lambda i: (i, 0)),
        out_shape=jax.ShapeDtypeStruct((NP, n), jnp.float32),
    )(acc, xs, b4)
    return out[:N_NODES]


def _combine2_body(acc_ref, xs_ref, o_ref):
    acc = acc_ref[...]
    num = jnp.concatenate([acc[0, :, :HC], acc[1, :, :HC]], axis=1)
    w = acc[0, :, HC:HC + 1]
    rs = jnp.where(w == 0.0, 1e-12, w)
    h = (w * xs_ref[...] + num) / rs
    o_ref[...] = jnp.where(h > 0, h, jnp.exp(jnp.minimum(h, 0.0)) - 1.0)


def _combine2(acc, xs, bm=512):
    xs = jnp.pad(xs, ((0, NP - N_NODES), (0, 0)))
    out = pl.pallas_call(
        _combine2_body,
        grid=(NP // bm,),
        in_specs=[pl.BlockSpec((2, bm, HROW), lambda i: (0, i, 0)),
                  pl.BlockSpec((bm, 128), lambda i: (i, 0))],
        out_specs=pl.BlockSpec((bm, 128), lambda i: (i, 0)),
        out_shape=jax.ShapeDtypeStruct((NP, 128), jnp.float32),
    )(acc, xs)
    return out[:N_NODES]


@functools.cache
def _make_edge_kernel(seq_er):
    ew1 = E1P // 16
    ew2 = E2P // 16
    ch1 = ew1 // C
    ch2 = ew2 // C
    mesh = plsc.VectorSubcoreMesh(core_axis_name="c", subcore_axis_name="s")

    def body(src1, dst1, era, src2, dst2, t0, t1, table, dstrow, sstab,
             acc_out,
             srcv, srcb, dstv, t0v, t1v, rowv, erv, erv2, ssv, contribv,
             accs, sem1, sem2, sem3, sem4):
        cid = lax.axis_index("c")
        sid = lax.axis_index("s")

        def zrow(r, _):
            for kk in range(HROW // 16):
                contribv[r, pl.ds(kk * 16, 16)] = jnp.zeros((16,), jnp.float32)
            return 0
        lax.fori_loop(0, C, zrow, 0)

        def zcp(i, _):
            pltpu.sync_copy(contribv, accs.at[pl.ds(sid * RPS + i * C, C)])
            return 0
        lax.fori_loop(0, RPS // C, zcp, 0)
        plsc.subcore_barrier()

        lanes = lax.iota(jnp.int32, 16)
        nbase = cid * N_NODES
        tbase = cid * TP

        def compute(two_tables):
            def erow(e, _):
                sel = erv[e, pl.ds(HC, 16)]
                if two_tables:
                    sel = sel + erv2[e, pl.ds(HC, 16)]
                p = ssv[e, pl.ds(0, 16)] + rowv[e, pl.ds(HC, 16)] + sel
                p = jnp.where(p > 0, p, ALPHA * p)
                wv = jnp.exp(-p)
                wtail = jnp.where(lanes == 0, wv, 0.0)
                w = jnp.sum(wtail)
                for j in range(HC // 16):
                    f = j * 16
                    v = rowv[e, pl.ds(f, 16)] + erv[e, pl.ds(f, 16)]
                    if two_tables:
                        v = v + erv2[e, pl.ds(f, 16)]
                    contribv[e, pl.ds(f, 16)] = v * w
                contribv[e, pl.ds(HC, 16)] = wtail
                return 0
            lax.fori_loop(0, C, erow, 0)

        def bias(ref, off):
            def grp(g, _):
                sl = pl.ds(g * 16, 16)
                ref[sl] = ref[sl] + off
                return 0
            lax.fori_loop(0, C // 16, grp, 0)

        def bias2(g, _):
            sl = pl.ds(g * 16, 16)
            srcb[sl] = srcv[sl] + nbase
            dstv[sl] = dstv[sl] + nbase
            return 0

        def chunk_a(i, _):
            base = sid * ew1 + i * C
            pltpu.sync_copy(src1.at[pl.ds(base, C)], srcv)
            pltpu.sync_copy(dst1.at[pl.ds(base, C)], dstv)
            lax.fori_loop(0, C // 16, bias2, 0)
            g1 = pltpu.async_copy(dstrow.at[dstv], rowv, sem1)
            g2 = pltpu.async_copy(sstab.at[srcb], ssv, sem2)
            if seq_er:
                pltpu.sync_copy(era.at[pl.ds(cid * E1P + base, C)], erv)
            else:
                pltpu.sync_copy(era.at[pl.ds(base, C)], t0v)
                bias(t0v, tbase)
                pltpu.async_copy(table.at[t0v], erv, sem3).wait()
            g1.wait()
            g2.wait()
            compute(False)
            pltpu.sync_copy(contribv, accs.at[srcv], add=True)
            return 0
        lax.fori_loop(0, ch1, chunk_a, 0)

        def chunk_b(i, _):
            base = sid * ew2 + i * C
            pltpu.sync_copy(src2.at[pl.ds(base, C)], srcv)
            pltpu.sync_copy(dst2.at[pl.ds(base, C)], dstv)
            pltpu.sync_copy(t0.at[pl.ds(base, C)], t0v)
            pltpu.sync_copy(t1.at[pl.ds(base, C)], t1v)
            lax.fori_loop(0, C // 16, bias2, 0)
            bias(t0v, tbase)
            bias(t1v, tbase)
            g1 = pltpu.async_copy(dstrow.at[dstv], rowv, sem1)
            g2 = pltpu.async_copy(sstab.at[srcb], ssv, sem2)
            g3 = pltpu.async_copy(table.at[t0v], erv, sem3)
            g4 = pltpu.async_copy(table.at[t1v], erv2, sem4)
            g1.wait()
            g2.wait()
            g3.wait()
            g4.wait()
            compute(True)
            pltpu.sync_copy(contribv, accs.at[srcv], add=True)
            return 0
        lax.fori_loop(0, ch2, chunk_b, 0)

        plsc.subcore_barrier()
        pltpu.sync_copy(accs.at[pl.ds(sid * RPS, RPS)],
                        acc_out.at[cid, pl.ds(sid * RPS, RPS)])

    return pl.kernel(
        body,
        out_type=jax.ShapeDtypeStruct((2, NP, HROW), jnp.float32),
        mesh=mesh,
        compiler_params=pltpu.CompilerParams(needs_layout_passes=False,
                                             use_tc_tiling_on_sc=False),
        scratch_types=[
            pltpu.VMEM((C,), jnp.int32),
            pltpu.VMEM((C,), jnp.int32),
            pltpu.VMEM((C,), jnp.int32),
            pltpu.VMEM((C,), jnp.int32),
            pltpu.VMEM((C,), jnp.int32),
            pltpu.VMEM((C, HROW), jnp.float32),
            pltpu.VMEM((C, HROW), jnp.float32),
            pltpu.VMEM((C, HROW), jnp.float32),
            pltpu.VMEM((C, SSW), jnp.float32),
            pltpu.VMEM((C, HROW), jnp.float32),
            pltpu.VMEM_SHARED((NP, HROW), jnp.float32),
            pltpu.SemaphoreType.DMA,
            pltpu.SemaphoreType.DMA,
            pltpu.SemaphoreType.DMA,
            pltpu.SemaphoreType.DMA,
        ],
    )


def _halves(payload, scal):
    m = payload.shape[0]
    z = jnp.zeros((m, HROW - HC - 1), payload.dtype)
    halves = [jnp.concatenate([payload[:, c * HC:(c + 1) * HC],
                               scal[:, c:c + 1], z], axis=1)
              for c in range(2)]
    return jnp.concatenate(halves, axis=0)


def kernel(Corpus_, entity_embeddings, relation_embed, edge_list, edge_type,
           edge_embed, edge_list_nhop, edge_type_nhop, a_heads, a2_heads,
           W, a_out, a2_out):
    f32 = jnp.float32
    x = entity_embeddings.astype(f32)
    rel = relation_embed.astype(f32)
    el = edge_list.astype(jnp.int32)
    et = edge_type.astype(jnp.int32)
    ee = edge_embed.astype(f32)
    eln = edge_list_nhop.astype(jnp.int32)
    etn = edge_type_nhop.astype(jnp.int32)
    a_heads = a_heads.astype(f32)
    a2_heads = a2_heads.astype(f32)

    a_s = a_heads[:, :, :NFEAT]
    a_d = a_heads[:, :, NFEAT:2 * NFEAT]
    a_r = a_heads[:, :, 2 * NFEAT:]
    a2s = jnp.einsum("hij,hjk->hk", a2_heads, a_s)
    a2d = jnp.einsum("hij,hjk->hk", a2_heads, a_d)
    a2r = jnp.einsum("hij,hjk->hk", a2_heads, a_r)

    as2 = a_out[:, :128].astype(f32)
    ad2 = a_out[:, 128:256].astype(f32)
    ar2 = a_out[:, 256:].astype(f32)
    a2o = a2_out.astype(f32)

    b1 = jnp.concatenate(
        [a_s[0].T, a_s[1].T, a_d[0].T, a_d[1].T,
         a2s.T, a2d.T, jnp.zeros((128, 124), f32)], axis=1)
    p1 = _mm(x, b1, bm=400)
    xs1 = p1[:, :128]
    dstrow1 = _halves(p1[:, 128:256], p1[:, 258:260])
    sstab1 = jnp.concatenate(
        [jnp.stack([p1[:, 256], p1[:, 257]])[:, :, None],
         jnp.zeros((2, N_NODES, SSW - 1), f32)],
        axis=2).reshape(2 * N_NODES, SSW)

    b2 = jnp.concatenate(
        [a_r[0].T, a_r[1].T, a2r.T, jnp.zeros((128, 14), f32)], axis=1)
    er1 = _mm(ee, b2, bm=640)
    pad_blk = jnp.zeros((E1P - E1, 130), f32).at[:, 128:130].set(1e30)
    er1 = jnp.concatenate([er1[:, :130], pad_blk], axis=0)
    er1 = _halves(er1[:, :128], er1[:, 128:130])

    tr1 = _mm(rel, b2, bm=256)
    dead1 = jnp.zeros((1, 130), f32).at[0, 128:130].set(1e30)
    tr1 = jnp.concatenate(
        [tr1[:, :130], dead1, jnp.zeros((TP - N_REL - 1, 130), f32)], axis=0)
    table1 = _halves(tr1[:, :128], tr1[:, 128:130])

    out_rel = _mm(rel, W.astype(f32), bm=256)

    b5 = jnp.concatenate(
        [ar2.T, (a2o @ ar2).T, jnp.zeros((128, 15), f32)], axis=1)
    t2 = _mm(out_rel, b5, bm=256)
    dead2 = jnp.zeros((1, 129), f32).at[0, 128:129].set(1e30)
    t2 = jnp.concatenate(
        [t2[:, :129], dead2, jnp.zeros((TP - N_REL - 1, 129), f32)], axis=0)
    table2 = _halves(t2[:, :128], t2[:, 128:129][:, [0, 0]])

    src1 = jnp.pad(el[0], (0, E1P - E1))
    dst1 = jnp.pad(el[1], (0, E1P - E1))
    ty1 = jnp.pad(et, (0, E1P - E1), constant_values=N_REL)
    src2 = jnp.pad(eln[0], (0, E2P - E2))
    dst2 = jnp.pad(eln[1], (0, E2P - E2))
    t0 = jnp.pad(etn[:, 0], (0, E2P - E2), constant_values=N_REL)
    t1 = jnp.pad(etn[:, 1], (0, E2P - E2), constant_values=N_REL)

    acc1 = _make_edge_kernel(True)(src1, dst1, er1, src2, dst2, t0, t1,
                                   table1, dstrow1, sstab1)

    b4 = jnp.concatenate(
        [as2.T, ad2.T, (a2o @ as2).T, (a2o @ ad2).T,
         jnp.zeros((128, 126), f32)], axis=1)
    p2 = _combine1(acc1, xs1, b4)
    xs2 = p2[:, :128]
    dstrow2 = _halves(p2[:, 128:256], p2[:, 257:258][:, [0, 0]])
    sstab2 = jnp.concatenate(
        [jnp.stack([p2[:, 256], p2[:, 256]])[:, :, None],
         jnp.zeros((2, N_NODES, SSW - 1), f32)],
        axis=2).reshape(2 * N_NODES, SSW)

    acc2 = _make_edge_kernel(False)(src1, dst1, ty1, src2, dst2, t0, t1,
                                    table2, dstrow2, sstab2)

    out = _combine2(acc2, xs2)
    return (out, out_rel)


# speedup vs baseline: 1.8029x; 1.2684x over previous
import functools

import jax
import jax.numpy as jnp
from jax import lax
from jax.experimental import pallas as pl
from jax.experimental.pallas import tpu as pltpu
from jax.experimental.pallas import tpu_sc as plsc

NHEADS = 2
ALPHA = 0.2
N_NODES = 10000
NFEAT = 128
NHID = 64
REL_DIM = 128
N_REL = 200
E1 = 160000
E2 = 32000

HC = 64
HROW = 80
SSW = 16
NW = 32
C = 128
E1P = 163840
E2P = 32768
TP = 208
NP = 10240
RPS = NP // 16


def _mm_body(x_ref, b_ref, o_ref):
    o_ref[...] = jnp.dot(x_ref[...], b_ref[...],
                         preferred_element_type=jnp.float32)


def _mm(x, b, bm):
    m, k = x.shape
    n = b.shape[1]
    mp = ((m + bm - 1) // bm) * bm
    if mp != m:
        x = jnp.pad(x, ((0, mp - m), (0, 0)))
    out = pl.pallas_call(
        _mm_body,
        grid=(mp // bm,),
        in_specs=[pl.BlockSpec((bm, k), lambda i: (i, 0)),
                  pl.BlockSpec((k, n), lambda i: (0, 0))],
        out_specs=pl.BlockSpec((bm, n), lambda i: (i, 0)),
        out_shape=jax.ShapeDtypeStruct((mp, n), jnp.float32),
    )(x, b)
    return out[:m] if mp != m else out


def _combine1_body(acc_ref, xs_ref, b4_ref, p2_ref):
    acc = acc_ref[...]
    num = jnp.concatenate([acc[0, :, :HC], acc[1, :, :HC]], axis=1)
    w = jnp.concatenate([acc[0, :, HC:HC + 1], acc[1, :, HC:HC + 1]], axis=1)
    rs = jnp.where(w == 0.0, 1e-12, w)
    bm = num.shape[0]
    wf = jnp.concatenate(
        [jnp.broadcast_to(w[:, h:h + 1], (bm, NHID)) for h in range(NHEADS)],
        axis=1)
    rsf = jnp.concatenate(
        [jnp.broadcast_to(rs[:, h:h + 1], (bm, NHID)) for h in range(NHEADS)],
        axis=1)
    h = (wf * xs_ref[...] + num) / rsf
    x2 = jnp.where(h > 0, h, jnp.exp(jnp.minimum(h, 0.0)) - 1.0)
    p2_ref[...] = jnp.dot(x2, b4_ref[...], preferred_element_type=jnp.float32)


def _combine1(acc, xs, b4, bm=512):
    n = b4.shape[1]
    xs = jnp.pad(xs, ((0, NP - N_NODES), (0, 0)))
    out = pl.pallas_call(
        _combine1_body,
        grid=(NP // bm,),
        in_specs=[pl.BlockSpec((2, bm, HROW), lambda i: (0, i, 0)),
                  pl.BlockSpec((bm, 128), lambda i: (i, 0)),
                  pl.BlockSpec((128, n), lambda i: (0, 0))],
        out_specs=pl.BlockSpec((bm, n), lambda i: (i, 0)),
        out_shape=jax.ShapeDtypeStruct((NP, n), jnp.float32),
    )(acc, xs, b4)
    return out[:N_NODES]


def _combine2_body(acc_ref, xs_ref, o_ref):
    acc = acc_ref[...]
    num = jnp.concatenate([acc[0, :, :HC], acc[1, :, :HC]], axis=1)
    w = acc[0, :, HC:HC + 1]
    rs = jnp.where(w == 0.0, 1e-12, w)
    h = (w * xs_ref[...] + num) / rs
    o_ref[...] = jnp.where(h > 0, h, jnp.exp(jnp.minimum(h, 0.0)) - 1.0)


def _combine2(acc, xs, bm=512):
    xs = jnp.pad(xs, ((0, NP - N_NODES), (0, 0)))
    out = pl.pallas_call(
        _combine2_body,
        grid=(NP // bm,),
        in_specs=[pl.BlockSpec((2, bm, HROW), lambda i: (0, i, 0)),
                  pl.BlockSpec((bm, 128), lambda i: (i, 0))],
        out_specs=pl.BlockSpec((bm, 128), lambda i: (i, 0)),
        out_shape=jax.ShapeDtypeStruct((NP, 128), jnp.float32),
    )(acc, xs)
    return out[:N_NODES]


@functools.cache
def _make_edge_kernel(seq_er):
    ew1 = E1P // 16
    ew2 = E2P // 16
    ch1 = ew1 // C
    ch2 = ew2 // C
    mesh = plsc.VectorSubcoreMesh(core_axis_name="c", subcore_axis_name="s")

    def body(src1, dst1, era, src2, dst2, t0, t1, table, dstrow, sstab,
             acc_out, *scr):
        (srcv, dstv, tv, t2v, srcs, srcb, rowv, ssv, erv, contribv) = (
            scr[0:2], scr[2:4], scr[4:6], scr[6:8], scr[8:10], scr[10:12],
            scr[12:14], scr[14:16], scr[16:18], scr[18:20])
        erv2 = scr[20]
        accs = scr[21]
        (semidx, semr, semss, seme, semsc) = (
            scr[22:24], scr[24:26], scr[26:28], scr[28:30], scr[30:32])
        seme2 = scr[32]
        cid = lax.axis_index("c")
        sid = lax.axis_index("s")

        def zrow(r, _):
            for kk in range(HROW // 16):
                contribv[0][r, pl.ds(kk * 16, 16)] = jnp.zeros((16,),
                                                               jnp.float32)
            return 0
        lax.fori_loop(0, C, zrow, 0)

        def zcp(i, _):
            pltpu.sync_copy(contribv[0], accs.at[pl.ds(sid * RPS + i * C, C)])
            return 0
        lax.fori_loop(0, RPS // C, zcp, 0)
        plsc.subcore_barrier()

        lanes = lax.iota(jnp.int32, 16)
        nbase = cid * N_NODES
        tbase = cid * TP

        def compute(b, two_tables):
            rowb, erb, er2b, ssb, ctb = (rowv[b], erv[b], erv2, ssv[b],
                                         contribv[b])

            def grp(g, _):
                rb = g * 16
                ridx = rb + lanes
                zc = jnp.zeros((16,), jnp.int32)
                cc = jnp.full((16,), HC, jnp.int32)
                ss16 = plsc.load_gather(ssb, [ridx, zc])
                sd16 = plsc.load_gather(rowb, [ridx, cc])
                se16 = plsc.load_gather(erb, [ridx, cc])
                if two_tables:
                    se16 = se16 + plsc.load_gather(er2b, [ridx, cc])
                p = ss16 + sd16 + se16
                p = jnp.where(p > 0, p, ALPHA * p)
                wv = jnp.exp(-p)

                def edge(j, _):
                    e = rb + j
                    wj = jnp.take_along_axis(
                        wv, jnp.full((16,), j, jnp.int32), axis=0,
                        mode="promise_in_bounds")
                    for k in range(HC // 16):
                        f = k * 16
                        v = rowb[e, pl.ds(f, 16)] + erb[e, pl.ds(f, 16)]
                        if two_tables:
                            v = v + er2b[e, pl.ds(f, 16)]
                        ctb[e, pl.ds(f, 16)] = v * wj
                    ctb[e, pl.ds(HC, 16)] = jnp.where(lanes == 0, wj, 0.0)
                    return 0
                lax.fori_loop(0, 16, edge, 0, unroll=4)
                return 0
            lax.fori_loop(0, C // 16, grp, 0)

        def wait_idx(b):
            pltpu.make_async_copy(src1.at[pl.ds(0, C)], srcv[b],
                                  semidx[b]).wait()

        def wait_row(b, sem):
            pltpu.make_async_copy(dstrow.at[pl.ds(0, C)], rowv[b], sem).wait()

        def drain_scatter(b):
            pltpu.make_async_copy(dstrow.at[pl.ds(0, C)], contribv[b],
                                  semsc[b]).wait()

        def finish(b, two_tables):
            wait_row(b, semr[b])
            pltpu.make_async_copy(sstab.at[pl.ds(0, C)], ssv[b],
                                  semss[b]).wait()
            wait_row(b, seme[b])
            if two_tables:
                pltpu.async_copy(table.at[t2v[b]], erv2, seme2).wait()
            compute(b, two_tables)
            pltpu.async_copy(contribv[b], accs.at[srcs[b]], semsc[b],
                             add=True)

        def make_phase(ch, ew, srca, dsta, n_idx, two_tables, a_tables,
                       seq_base):
            def issue_idx(i, b):
                base = sid * ew + i * C
                pltpu.async_copy(srca.at[pl.ds(base, C)], srcv[b], semidx[b])
                pltpu.async_copy(dsta.at[pl.ds(base, C)], dstv[b], semidx[b])
                if a_tables >= 1:
                    ta = era if a_tables == 1 and not two_tables else t0
                    pltpu.async_copy(ta.at[pl.ds(base, C)], tv[b], semidx[b])
                if a_tables == 2:
                    pltpu.async_copy(t1.at[pl.ds(base, C)], t2v[b], semidx[b])

            def launch(i, b):
                for _ in range(n_idx):
                    wait_idx(b)

                def bias_grp(g, _):
                    sl = pl.ds(g * 16, 16)
                    s = srcv[b][sl]
                    srcs[b][sl] = s
                    srcb[b][sl] = s + nbase
                    dstv[b][sl] = dstv[b][sl] + nbase
                    if a_tables >= 1:
                        tv[b][sl] = tv[b][sl] + tbase
                    if a_tables == 2:
                        t2v[b][sl] = t2v[b][sl] + tbase
                    return 0
                lax.fori_loop(0, C // 16, bias_grp, 0)
                pltpu.async_copy(dstrow.at[dstv[b]], rowv[b], semr[b])
                pltpu.async_copy(sstab.at[srcb[b]], ssv[b], semss[b])
                if seq_base is not None:
                    base = seq_base + sid * ew + i * C
                    pltpu.async_copy(era.at[pl.ds(base, C)], erv[b], seme[b])
                else:
                    pltpu.async_copy(table.at[tv[b]], erv[b], seme[b])

            def run():
                issue_idx(0, 0)
                issue_idx(1, 1)
                launch(0, 0)

                def step(i2, _):
                    for b in (0, 1):
                        i = i2 * 2 + b
                        if b == 1:
                            drain_scatter(0)
                        else:
                            @pl.when(i2 >= 1)
                            def _():
                                drain_scatter(1)
                        if b == 0:
                            launch(i + 1, 1)
                        else:
                            @pl.when(i2 < ch // 2 - 1)
                            def _():
                                launch(i + 1, 0)
                        finish(b, two_tables)

                        @pl.when(i2 < ch // 2 - 1)
                        def _():
                            issue_idx(i + 2, b)
                    return 0
                lax.fori_loop(0, ch // 2, step, 0)
                drain_scatter(1)
            return run

        if seq_er:
            make_phase(ch1, ew1, src1, dst1, 2, False, 0, cid * E1P)()
        else:
            make_phase(ch1, ew1, src1, dst1, 3, False, 1, None)()
        make_phase(ch2, ew2, src2, dst2, 4, True, 2, None)()

        plsc.subcore_barrier()
        pltpu.sync_copy(accs.at[pl.ds(sid * RPS, RPS)],
                        acc_out.at[cid, pl.ds(sid * RPS, RPS)])

    idx_t = pltpu.VMEM((C,), jnp.int32)
    row_t = pltpu.VMEM((C, HROW), jnp.float32)
    return pl.kernel(
        body,
        out_type=jax.ShapeDtypeStruct((2, NP, HROW), jnp.float32),
        mesh=mesh,
        compiler_params=pltpu.CompilerParams(needs_layout_passes=False,
                                             use_tc_tiling_on_sc=False),
        scratch_types=(
            [idx_t] * 12
            + [row_t, row_t]
            + [pltpu.VMEM((C, SSW), jnp.float32)] * 2
            + [row_t, row_t]
            + [row_t, row_t]
            + [row_t]
            + [pltpu.VMEM_SHARED((NP, HROW), jnp.float32)]
            + [pltpu.SemaphoreType.DMA] * 11
        ),
    )


def _halves(payload, scal):
    m = payload.shape[0]
    z = jnp.zeros((m, HROW - HC - 1), payload.dtype)
    halves = [jnp.concatenate([payload[:, c * HC:(c + 1) * HC],
                               scal[:, c:c + 1], z], axis=1)
              for c in range(2)]
    return jnp.concatenate(halves, axis=0)


def kernel(Corpus_, entity_embeddings, relation_embed, edge_list, edge_type,
           edge_embed, edge_list_nhop, edge_type_nhop, a_heads, a2_heads,
           W, a_out, a2_out):
    f32 = jnp.float32
    x = entity_embeddings.astype(f32)
    rel = relation_embed.astype(f32)
    el = edge_list.astype(jnp.int32)
    et = edge_type.astype(jnp.int32)
    ee = edge_embed.astype(f32)
    eln = edge_list_nhop.astype(jnp.int32)
    etn = edge_type_nhop.astype(jnp.int32)
    a_heads = a_heads.astype(f32)
    a2_heads = a2_heads.astype(f32)

    a_s = a_heads[:, :, :NFEAT]
    a_d = a_heads[:, :, NFEAT:2 * NFEAT]
    a_r = a_heads[:, :, 2 * NFEAT:]
    a2s = jnp.einsum("hij,hjk->hk", a2_heads, a_s)
    a2d = jnp.einsum("hij,hjk->hk", a2_heads, a_d)
    a2r = jnp.einsum("hij,hjk->hk", a2_heads, a_r)

    as2 = a_out[:, :128].astype(f32)
    ad2 = a_out[:, 128:256].astype(f32)
    ar2 = a_out[:, 256:].astype(f32)
    a2o = a2_out.astype(f32)

    b1 = jnp.concatenate(
        [a_s[0].T, a_s[1].T, a_d[0].T, a_d[1].T,
         a2s.T, a2d.T, jnp.zeros((128, 124), f32)], axis=1)
    p1 = _mm(x, b1, bm=400)
    xs1 = p1[:, :128]
    dstrow1 = _halves(p1[:, 128:256], p1[:, 258:260])
    sstab1 = jnp.concatenate(
        [jnp.stack([p1[:, 256], p1[:, 257]])[:, :, None],
         jnp.zeros((2, N_NODES, SSW - 1), f32)],
        axis=2).reshape(2 * N_NODES, SSW)

    b2 = jnp.concatenate(
        [a_r[0].T, a_r[1].T, a2r.T, jnp.zeros((128, 14), f32)], axis=1)
    er1 = _mm(ee, b2, bm=640)
    pad_blk = jnp.zeros((E1P - E1, 130), f32).at[:, 128:130].set(1e30)
    er1 = jnp.concatenate([er1[:, :130], pad_blk], axis=0)
    er1 = _halves(er1[:, :128], er1[:, 128:130])

    tr1 = _mm(rel, b2, bm=256)
    dead1 = jnp.zeros((1, 130), f32).at[0, 128:130].set(1e30)
    tr1 = jnp.concatenate(
        [tr1[:, :130], dead1, jnp.zeros((TP - N_REL - 1, 130), f32)], axis=0)
    table1 = _halves(tr1[:, :128], tr1[:, 128:130])

    out_rel = _mm(rel, W.astype(f32), bm=256)

    b5 = jnp.concatenate(
        [ar2.T, (a2o @ ar2).T, jnp.zeros((128, 15), f32)], axis=1)
    t2 = _mm(out_rel, b5, bm=256)
    dead2 = jnp.zeros((1, 129), f32).at[0, 128:129].set(1e30)
    t2 = jnp.concatenate(
        [t2[:, :129], dead2, jnp.zeros((TP - N_REL - 1, 129), f32)], axis=0)
    table2 = _halves(t2[:, :128], t2[:, 128:129][:, [0, 0]])

    src1 = jnp.pad(el[0], (0, E1P - E1))
    dst1 = jnp.pad(el[1], (0, E1P - E1))
    ty1 = jnp.pad(et, (0, E1P - E1), constant_values=N_REL)
    src2 = jnp.pad(eln[0], (0, E2P - E2))
    dst2 = jnp.pad(eln[1], (0, E2P - E2))
    t0 = jnp.pad(etn[:, 0], (0, E2P - E2), constant_values=N_REL)
    t1 = jnp.pad(etn[:, 1], (0, E2P - E2), constant_values=N_REL)

    acc1 = _make_edge_kernel(True)(src1, dst1, er1, src2, dst2, t0, t1,
                                   table1, dstrow1, sstab1)

    b4 = jnp.concatenate(
        [as2.T, ad2.T, (a2o @ as2).T, (a2o @ ad2).T,
         jnp.zeros((128, 126), f32)], axis=1)
    p2 = _combine1(acc1, xs1, b4)
    xs2 = p2[:, :128]
    dstrow2 = _halves(p2[:, 128:256], p2[:, 257:258][:, [0, 0]])
    sstab2 = jnp.concatenate(
        [jnp.stack([p2[:, 256], p2[:, 256]])[:, :, None],
         jnp.zeros((2, N_NODES, SSW - 1), f32)],
        axis=2).reshape(2 * N_NODES, SSW)

    acc2 = _make_edge_kernel(False)(src1, dst1, ty1, src2, dst2, t0, t1,
                                    table2, dstrow2, sstab2)

    out = _combine2(acc2, xs2)
    return (out, out_rel)


# speedup vs baseline: 2.1607x; 1.1985x over previous
import functools

import jax
import jax.numpy as jnp
from jax import lax
from jax.experimental import pallas as pl
from jax.experimental.pallas import tpu as pltpu
from jax.experimental.pallas import tpu_sc as plsc

NHEADS = 2
ALPHA = 0.2
N_NODES = 10000
NFEAT = 128
NHID = 64
REL_DIM = 128
N_REL = 200
E1 = 160000
E2 = 32000

HC = 64
HROW = 80
SSW = 16
NW = 32
C = 128
E1P = 163840
E2P = 32768
TP = 208
NP = 10240
RPS = NP // 16


def _mm_body(x_ref, b_ref, o_ref):
    o_ref[...] = jnp.dot(x_ref[...], b_ref[...],
                         preferred_element_type=jnp.float32)


def _mm(x, b, bm):
    m, k = x.shape
    n = b.shape[1]
    mp = ((m + bm - 1) // bm) * bm
    if mp != m:
        x = jnp.pad(x, ((0, mp - m), (0, 0)))
    out = pl.pallas_call(
        _mm_body,
        grid=(mp // bm,),
        in_specs=[pl.BlockSpec((bm, k), lambda i: (i, 0)),
                  pl.BlockSpec((k, n), lambda i: (0, 0))],
        out_specs=pl.BlockSpec((bm, n), lambda i: (i, 0)),
        out_shape=jax.ShapeDtypeStruct((mp, n), jnp.float32),
    )(x, b)
    return out[:m] if mp != m else out


_ER_BM = 640


def _er_body(ee_ref, b_ref, o_ref):
    j = pl.program_id(1)
    y = jnp.dot(ee_ref[...], b_ref[0], preferred_element_type=jnp.float32)
    rowid = j * _ER_BM + lax.broadcasted_iota(jnp.int32, y.shape, 0)
    colid = lax.broadcasted_iota(jnp.int32, y.shape, 1)
    pad = jnp.where(colid == HC, jnp.float32(1e30), jnp.float32(0.0))
    o_ref[0] = jnp.where(rowid >= E1, pad, y)


def _er_tables(ee, b):
    nb = E1 // _ER_BM
    out = pl.pallas_call(
        _er_body,
        grid=(2, E1P // _ER_BM),
        in_specs=[
            pl.BlockSpec((_ER_BM, 128),
                         lambda c, j: (jnp.minimum(j, nb - 1), 0)),
            pl.BlockSpec((1, 128, HROW), lambda c, j: (c, 0, 0)),
        ],
        out_specs=pl.BlockSpec((1, _ER_BM, HROW), lambda c, j: (c, j, 0)),
        out_shape=jax.ShapeDtypeStruct((2, E1P, HROW), jnp.float32),
    )(ee, b)
    return out.reshape(2 * E1P, HROW)


def _combine1_body(acc_ref, xs_ref, b4_ref, p2_ref):
    acc = acc_ref[...]
    num = jnp.concatenate([acc[0, :, :HC], acc[1, :, :HC]], axis=1)
    w = jnp.concatenate([acc[0, :, HC:HC + 1], acc[1, :, HC:HC + 1]], axis=1)
    rs = jnp.where(w == 0.0, 1e-12, w)
    bm = num.shape[0]
    wf = jnp.concatenate(
        [jnp.broadcast_to(w[:, h:h + 1], (bm, NHID)) for h in range(NHEADS)],
        axis=1)
    rsf = jnp.concatenate(
        [jnp.broadcast_to(rs[:, h:h + 1], (bm, NHID)) for h in range(NHEADS)],
        axis=1)
    h = (wf * xs_ref[...] + num) / rsf
    x2 = jnp.where(h > 0, h, jnp.exp(jnp.minimum(h, 0.0)) - 1.0)
    p2_ref[...] = jnp.dot(x2, b4_ref[...], preferred_element_type=jnp.float32)


def _combine1(acc, xs, b4, bm=512):
    n = b4.shape[1]
    xs = jnp.pad(xs, ((0, NP - N_NODES), (0, 0)))
    out = pl.pallas_call(
        _combine1_body,
        grid=(NP // bm,),
        in_specs=[pl.BlockSpec((2, bm, HROW), lambda i: (0, i, 0)),
                  pl.BlockSpec((bm, 128), lambda i: (i, 0)),
                  pl.BlockSpec((128, n), lambda i: (0, 0))],
        out_specs=pl.BlockSpec((bm, n), lambda i: (i, 0)),
        out_shape=jax.ShapeDtypeStruct((NP, n), jnp.float32),
    )(acc, xs, b4)
    return out[:N_NODES]


def _combine2_body(acc_ref, xs_ref, o_ref):
    acc = acc_ref[...]
    num = jnp.concatenate([acc[0, :, :HC], acc[1, :, :HC]], axis=1)
    w = acc[0, :, HC:HC + 1]
    rs = jnp.where(w == 0.0, 1e-12, w)
    h = (w * xs_ref[...] + num) / rs
    o_ref[...] = jnp.where(h > 0, h, jnp.exp(jnp.minimum(h, 0.0)) - 1.0)


def _combine2(acc, xs, bm=512):
    xs = jnp.pad(xs, ((0, NP - N_NODES), (0, 0)))
    out = pl.pallas_call(
        _combine2_body,
        grid=(NP // bm,),
        in_specs=[pl.BlockSpec((2, bm, HROW), lambda i: (0, i, 0)),
                  pl.BlockSpec((bm, 128), lambda i: (i, 0))],
        out_specs=pl.BlockSpec((bm, 128), lambda i: (i, 0)),
        out_shape=jax.ShapeDtypeStruct((NP, 128), jnp.float32),
    )(acc, xs)
    return out[:N_NODES]


@functools.cache
def _make_edge_kernel(seq_er):
    ew1 = E1P // 16
    ew2 = E2P // 16
    ch1 = ew1 // C
    ch2 = ew2 // C
    mesh = plsc.VectorSubcoreMesh(core_axis_name="c", subcore_axis_name="s")

    def body(src1, dst1, era, src2, dst2, t0, t1, table, dstrow, sstab,
             acc_out, *scr):
        (srcv, dstv, tv, t2v, srcs, srcb, rowv, ssv, erv, contribv) = (
            scr[0:2], scr[2:4], scr[4:6], scr[6:8], scr[8:10], scr[10:12],
            scr[12:14], scr[14:16], scr[16:18], scr[18:20])
        erv2 = scr[20]
        accs = scr[21]
        (semidx, semr, semss, seme, semsc) = (
            scr[22:24], scr[24:26], scr[26:28], scr[28:30], scr[30:32])
        seme2 = scr[32]
        cid = lax.axis_index("c")
        sid = lax.axis_index("s")

        def zrow(r, _):
            for kk in range(HROW // 16):
                contribv[0][r, pl.ds(kk * 16, 16)] = jnp.zeros((16,),
                                                               jnp.float32)
            return 0
        lax.fori_loop(0, C, zrow, 0)

        def zcp(i, _):
            pltpu.sync_copy(contribv[0], accs.at[pl.ds(sid * RPS + i * C, C)])
            return 0
        lax.fori_loop(0, RPS // C, zcp, 0)
        plsc.subcore_barrier()

        lanes = lax.iota(jnp.int32, 16)
        nbase = cid * N_NODES
        tbase = cid * TP

        def compute(b, two_tables):
            rowb, erb, er2b, ssb, ctb = (rowv[b], erv[b], erv2, ssv[b],
                                         contribv[b])

            def grp(g, _):
                rb = g * 16
                ridx = rb + lanes
                zc = jnp.zeros((16,), jnp.int32)
                cc = jnp.full((16,), HC, jnp.int32)
                ss16 = plsc.load_gather(ssb, [ridx, zc])
                sd16 = plsc.load_gather(rowb, [ridx, cc])
                se16 = plsc.load_gather(erb, [ridx, cc])
                if two_tables:
                    se16 = se16 + plsc.load_gather(er2b, [ridx, cc])
                p = ss16 + sd16 + se16
                p = jnp.where(p > 0, p, ALPHA * p)
                wv = jnp.exp(-p)

                def edge(j, _):
                    e = rb + j
                    wj = jnp.take_along_axis(
                        wv, jnp.full((16,), j, jnp.int32), axis=0,
                        mode="promise_in_bounds")
                    for k in range(HC // 16):
                        f = k * 16
                        v = rowb[e, pl.ds(f, 16)] + erb[e, pl.ds(f, 16)]
                        if two_tables:
                            v = v + er2b[e, pl.ds(f, 16)]
                        ctb[e, pl.ds(f, 16)] = v * wj
                    ctb[e, pl.ds(HC, 16)] = jnp.where(lanes == 0, wj, 0.0)
                    return 0
                lax.fori_loop(0, 16, edge, 0, unroll=4)
                return 0
            lax.fori_loop(0, C // 16, grp, 0)

        def wait_idx(b):
            pltpu.make_async_copy(src1.at[pl.ds(0, C)], srcv[b],
                                  semidx[b]).wait()

        def wait_row(b, sem):
            pltpu.make_async_copy(dstrow.at[pl.ds(0, C)], rowv[b], sem).wait()

        def drain_scatter(b):
            pltpu.make_async_copy(dstrow.at[pl.ds(0, C)], contribv[b],
                                  semsc[b]).wait()

        def finish(b, two_tables):
            wait_row(b, semr[b])
            pltpu.make_async_copy(sstab.at[pl.ds(0, C)], ssv[b],
                                  semss[b]).wait()
            wait_row(b, seme[b])
            if two_tables:
                pltpu.async_copy(table.at[t2v[b]], erv2, seme2).wait()
            compute(b, two_tables)
            pltpu.async_copy(contribv[b], accs.at[srcs[b]], semsc[b],
                             add=True)

        def make_phase(ch, ew, srca, dsta, n_idx, two_tables, a_tables,
                       seq_base):
            def issue_idx(i, b):
                base = sid * ew + i * C
                pltpu.async_copy(srca.at[pl.ds(base, C)], srcv[b], semidx[b])
                pltpu.async_copy(dsta.at[pl.ds(base, C)], dstv[b], semidx[b])
                if a_tables >= 1:
                    ta = era if a_tables == 1 and not two_tables else t0
                    pltpu.async_copy(ta.at[pl.ds(base, C)], tv[b], semidx[b])
                if a_tables == 2:
                    pltpu.async_copy(t1.at[pl.ds(base, C)], t2v[b], semidx[b])

            def launch(i, b):
                for _ in range(n_idx):
                    wait_idx(b)

                def bias_grp(g, _):
                    sl = pl.ds(g * 16, 16)
                    s = srcv[b][sl]
                    srcs[b][sl] = s
                    srcb[b][sl] = s + nbase
                    dstv[b][sl] = dstv[b][sl] + nbase
                    if a_tables >= 1:
                        tv[b][sl] = tv[b][sl] + tbase
                    if a_tables == 2:
                        t2v[b][sl] = t2v[b][sl] + tbase
                    return 0
                lax.fori_loop(0, C // 16, bias_grp, 0)
                pltpu.async_copy(dstrow.at[dstv[b]], rowv[b], semr[b])
                pltpu.async_copy(sstab.at[srcb[b]], ssv[b], semss[b])
                if seq_base is not None:
                    base = seq_base + sid * ew + i * C
                    pltpu.async_copy(era.at[pl.ds(base, C)], erv[b], seme[b])
                else:
                    pltpu.async_copy(table.at[tv[b]], erv[b], seme[b])

            def run():
                issue_idx(0, 0)
                issue_idx(1, 1)
                launch(0, 0)

                def step(i2, _):
                    for b in (0, 1):
                        i = i2 * 2 + b
                        if b == 1:
                            drain_scatter(0)
                        else:
                            @pl.when(i2 >= 1)
                            def _():
                                drain_scatter(1)
                        if b == 0:
                            launch(i + 1, 1)
                        else:
                            @pl.when(i2 < ch // 2 - 1)
                            def _():
                                launch(i + 1, 0)
                        finish(b, two_tables)

                        @pl.when(i2 < ch // 2 - 1)
                        def _():
                            issue_idx(i + 2, b)
                    return 0
                lax.fori_loop(0, ch // 2, step, 0)
                drain_scatter(1)
            return run

        if seq_er:
            make_phase(ch1, ew1, src1, dst1, 2, False, 0, cid * E1P)()
        else:
            make_phase(ch1, ew1, src1, dst1, 3, False, 1, None)()
        make_phase(ch2, ew2, src2, dst2, 4, True, 2, None)()

        plsc.subcore_barrier()
        pltpu.sync_copy(accs.at[pl.ds(sid * RPS, RPS)],
                        acc_out.at[cid, pl.ds(sid * RPS, RPS)])

    idx_t = pltpu.VMEM((C,), jnp.int32)
    row_t = pltpu.VMEM((C, HROW), jnp.float32)
    return pl.kernel(
        body,
        out_type=jax.ShapeDtypeStruct((2, NP, HROW), jnp.float32),
        mesh=mesh,
        compiler_params=pltpu.CompilerParams(needs_layout_passes=False,
                                             use_tc_tiling_on_sc=False),
        scratch_types=(
            [idx_t] * 12
            + [row_t, row_t]
            + [pltpu.VMEM((C, SSW), jnp.float32)] * 2
            + [row_t, row_t]
            + [row_t, row_t]
            + [row_t]
            + [pltpu.VMEM_SHARED((NP, HROW), jnp.float32)]
            + [pltpu.SemaphoreType.DMA] * 11
        ),
    )


def _halves(payload, scal):
    m = payload.shape[0]
    z = jnp.zeros((m, HROW - HC - 1), payload.dtype)
    halves = [jnp.concatenate([payload[:, c * HC:(c + 1) * HC],
                               scal[:, c:c + 1], z], axis=1)
              for c in range(2)]
    return jnp.concatenate(halves, axis=0)


def kernel(Corpus_, entity_embeddings, relation_embed, edge_list, edge_type,
           edge_embed, edge_list_nhop, edge_type_nhop, a_heads, a2_heads,
           W, a_out, a2_out):
    f32 = jnp.float32
    x = entity_embeddings.astype(f32)
    rel = relation_embed.astype(f32)
    el = edge_list.astype(jnp.int32)
    et = edge_type.astype(jnp.int32)
    ee = edge_embed.astype(f32)
    eln = edge_list_nhop.astype(jnp.int32)
    etn = edge_type_nhop.astype(jnp.int32)
    a_heads = a_heads.astype(f32)
    a2_heads = a2_heads.astype(f32)

    a_s = a_heads[:, :, :NFEAT]
    a_d = a_heads[:, :, NFEAT:2 * NFEAT]
    a_r = a_heads[:, :, 2 * NFEAT:]
    a2s = jnp.einsum("hij,hjk->hk", a2_heads, a_s)
    a2d = jnp.einsum("hij,hjk->hk", a2_heads, a_d)
    a2r = jnp.einsum("hij,hjk->hk", a2_heads, a_r)

    as2 = a_out[:, :128].astype(f32)
    ad2 = a_out[:, 128:256].astype(f32)
    ar2 = a_out[:, 256:].astype(f32)
    a2o = a2_out.astype(f32)

    b1 = jnp.concatenate(
        [a_s[0].T, a_s[1].T, a_d[0].T, a_d[1].T,
         a2s.T, a2d.T, jnp.zeros((128, 124), f32)], axis=1)
    p1 = _mm(x, b1, bm=400)
    xs1 = p1[:, :128]
    dstrow1 = _halves(p1[:, 128:256], p1[:, 258:260])
    sstab1 = jnp.concatenate(
        [jnp.stack([p1[:, 256], p1[:, 257]])[:, :, None],
         jnp.zeros((2, N_NODES, SSW - 1), f32)],
        axis=2).reshape(2 * N_NODES, SSW)

    zc = jnp.zeros((128, HROW - HC - 1), f32)
    b2h = jnp.stack([jnp.concatenate([a_r[c].T, a2r[c:c + 1].T, zc], axis=1)
                     for c in range(2)])
    er1 = _er_tables(ee, b2h)

    b2 = jnp.concatenate(
        [a_r[0].T, a_r[1].T, a2r.T, jnp.zeros((128, 14), f32)], axis=1)
    tr1 = _mm(rel, b2, bm=256)
    dead1 = jnp.zeros((1, 130), f32).at[0, 128:130].set(1e30)
    tr1 = jnp.concatenate(
        [tr1[:, :130], dead1, jnp.zeros((TP - N_REL - 1, 130), f32)], axis=0)
    table1 = _halves(tr1[:, :128], tr1[:, 128:130])

    out_rel = _mm(rel, W.astype(f32), bm=256)

    b5 = jnp.concatenate(
        [ar2.T, (a2o @ ar2).T, jnp.zeros((128, 15), f32)], axis=1)
    t2 = _mm(out_rel, b5, bm=256)
    dead2 = jnp.zeros((1, 129), f32).at[0, 128:129].set(1e30)
    t2 = jnp.concatenate(
        [t2[:, :129], dead2, jnp.zeros((TP - N_REL - 1, 129), f32)], axis=0)
    table2 = _halves(t2[:, :128], t2[:, 128:129][:, [0, 0]])

    src1 = jnp.pad(el[0], (0, E1P - E1))
    dst1 = jnp.pad(el[1], (0, E1P - E1))
    ty1 = jnp.pad(et, (0, E1P - E1), constant_values=N_REL)
    src2 = jnp.pad(eln[0], (0, E2P - E2))
    dst2 = jnp.pad(eln[1], (0, E2P - E2))
    t0 = jnp.pad(etn[:, 0], (0, E2P - E2), constant_values=N_REL)
    t1 = jnp.pad(etn[:, 1], (0, E2P - E2), constant_values=N_REL)

    acc1 = _make_edge_kernel(True)(src1, dst1, er1, src2, dst2, t0, t1,
                                   table1, dstrow1, sstab1)

    b4 = jnp.concatenate(
        [as2.T, ad2.T, (a2o @ as2).T, (a2o @ ad2).T,
         jnp.zeros((128, 126), f32)], axis=1)
    p2 = _combine1(acc1, xs1, b4)
    xs2 = p2[:, :128]
    dstrow2 = _halves(p2[:, 128:256], p2[:, 257:258][:, [0, 0]])
    sstab2 = jnp.concatenate(
        [jnp.stack([p2[:, 256], p2[:, 256]])[:, :, None],
         jnp.zeros((2, N_NODES, SSW - 1), f32)],
        axis=2).reshape(2 * N_NODES, SSW)

    acc2 = _make_edge_kernel(False)(src1, dst1, ty1, src2, dst2, t0, t1,
                                    table2, dstrow2, sstab2)

    out = _combine2(acc2, xs2)
    return (out, out_rel)


# speedup vs baseline: 2.4746x; 1.1453x over previous
import functools

import jax
import jax.numpy as jnp
from jax import lax
from jax.experimental import pallas as pl
from jax.experimental.pallas import tpu as pltpu
from jax.experimental.pallas import tpu_sc as plsc

NHEADS = 2
ALPHA = 0.2
N_NODES = 10000
NFEAT = 128
NHID = 64
REL_DIM = 128
N_REL = 200
E1 = 160000
E2 = 32000

HC = 64
HROW = 80
SSW = 16
NW = 32
C = 128
E1P = 163840
E2P = 32768
TP = 208
NP = 10240
RPS = NP // 16


def _mm_body(x_ref, b_ref, o_ref):
    o_ref[...] = jnp.dot(x_ref[...], b_ref[...],
                         preferred_element_type=jnp.float32)


def _mm(x, b, bm):
    m, k = x.shape
    n = b.shape[1]
    mp = ((m + bm - 1) // bm) * bm
    if mp != m:
        x = jnp.pad(x, ((0, mp - m), (0, 0)))
    out = pl.pallas_call(
        _mm_body,
        grid=(mp // bm,),
        in_specs=[pl.BlockSpec((bm, k), lambda i: (i, 0)),
                  pl.BlockSpec((k, n), lambda i: (0, 0))],
        out_specs=pl.BlockSpec((bm, n), lambda i: (i, 0)),
        out_shape=jax.ShapeDtypeStruct((mp, n), jnp.float32),
    )(x, b)
    return out[:m] if mp != m else out


_ER_BM = 640


def _er_body(ee_ref, b_ref, o_ref):
    j = pl.program_id(1)
    y = jnp.dot(ee_ref[...], b_ref[0], preferred_element_type=jnp.float32)
    rowid = j * _ER_BM + lax.broadcasted_iota(jnp.int32, y.shape, 0)
    colid = lax.broadcasted_iota(jnp.int32, y.shape, 1)
    pad = jnp.where(colid == HC, jnp.float32(1e30), jnp.float32(0.0))
    o_ref[0] = jnp.where(rowid >= E1, pad, y)


def _er_tables(ee, b):
    nb = E1 // _ER_BM
    out = pl.pallas_call(
        _er_body,
        grid=(2, E1P // _ER_BM),
        in_specs=[
            pl.BlockSpec((_ER_BM, 128),
                         lambda c, j: (jnp.minimum(j, nb - 1), 0)),
            pl.BlockSpec((1, 128, HROW), lambda c, j: (c, 0, 0)),
        ],
        out_specs=pl.BlockSpec((1, _ER_BM, HROW), lambda c, j: (c, j, 0)),
        out_shape=jax.ShapeDtypeStruct((2, E1P, HROW), jnp.float32),
    )(ee, b)
    return out.reshape(2 * E1P, HROW)


def _combine1_body(acc_ref, xs_ref, b4_ref, p2_ref):
    acc = acc_ref[...]
    num = jnp.concatenate([acc[0, :, :HC], acc[1, :, :HC]], axis=1)
    w = jnp.concatenate([acc[0, :, HC:HC + 1], acc[1, :, HC:HC + 1]], axis=1)
    rs = jnp.where(w == 0.0, 1e-12, w)
    bm = num.shape[0]
    wf = jnp.concatenate(
        [jnp.broadcast_to(w[:, h:h + 1], (bm, NHID)) for h in range(NHEADS)],
        axis=1)
    rsf = jnp.concatenate(
        [jnp.broadcast_to(rs[:, h:h + 1], (bm, NHID)) for h in range(NHEADS)],
        axis=1)
    h = (wf * xs_ref[...] + num) / rsf
    x2 = jnp.where(h > 0, h, jnp.exp(jnp.minimum(h, 0.0)) - 1.0)
    p2_ref[...] = jnp.dot(x2, b4_ref[...], preferred_element_type=jnp.float32)


def _combine1(acc, xs, b4, bm=512):
    n = b4.shape[1]
    xs = jnp.pad(xs, ((0, NP - N_NODES), (0, 0)))
    out = pl.pallas_call(
        _combine1_body,
        grid=(NP // bm,),
        in_specs=[pl.BlockSpec((2, bm, HROW), lambda i: (0, i, 0)),
                  pl.BlockSpec((bm, 128), lambda i: (i, 0)),
                  pl.BlockSpec((128, n), lambda i: (0, 0))],
        out_specs=pl.BlockSpec((bm, n), lambda i: (i, 0)),
        out_shape=jax.ShapeDtypeStruct((NP, n), jnp.float32),
    )(acc, xs, b4)
    return out[:N_NODES]


def _combine2_body(acc_ref, xs_ref, o_ref):
    acc = acc_ref[...]
    num = jnp.concatenate([acc[0, :, :HC], acc[1, :, :HC]], axis=1)
    w = acc[0, :, HC:HC + 1]
    rs = jnp.where(w == 0.0, 1e-12, w)
    h = (w * xs_ref[...] + num) / rs
    o_ref[...] = jnp.where(h > 0, h, jnp.exp(jnp.minimum(h, 0.0)) - 1.0)


def _combine2(acc, xs, bm=512):
    xs = jnp.pad(xs, ((0, NP - N_NODES), (0, 0)))
    out = pl.pallas_call(
        _combine2_body,
        grid=(NP // bm,),
        in_specs=[pl.BlockSpec((2, bm, HROW), lambda i: (0, i, 0)),
                  pl.BlockSpec((bm, 128), lambda i: (i, 0))],
        out_specs=pl.BlockSpec((bm, 128), lambda i: (i, 0)),
        out_shape=jax.ShapeDtypeStruct((NP, 128), jnp.float32),
    )(acc, xs)
    return out[:N_NODES]


@functools.cache
def _make_edge_kernel(seq_er):
    ew1 = E1P // 16
    ew2 = E2P // 16
    ch1 = ew1 // C
    ch2 = ew2 // C
    mesh = plsc.VectorSubcoreMesh(core_axis_name="c", subcore_axis_name="s")

    def body(src1, dst1, era, src2, dst2, t0, t1, table, dstrow, sstab,
             acc_out, *scr):
        (srcv, dstv, tv, t2v, srcs, srcb, rowv, ssv, erv, contribv) = (
            scr[0:2], scr[2:4], scr[4:6], scr[6:8], scr[8:10], scr[10:12],
            scr[12:14], scr[14:16], scr[16:18], scr[18:20])
        erv2 = scr[20]
        accs = scr[21]
        (semidx, semr, semss, seme, semsc) = (
            scr[22:24], scr[24:26], scr[26:28], scr[28:30], scr[30:32])
        seme2 = scr[32]
        cid = lax.axis_index("c")
        sid = lax.axis_index("s")

        def zrow(r, _):
            for kk in range(HROW // 16):
                contribv[0][r, pl.ds(kk * 16, 16)] = jnp.zeros((16,),
                                                               jnp.float32)
            return 0
        lax.fori_loop(0, C, zrow, 0)

        def zcp(i, _):
            pltpu.sync_copy(contribv[0], accs.at[pl.ds(sid * RPS + i * C, C)])
            return 0
        lax.fori_loop(0, RPS // C, zcp, 0)
        plsc.subcore_barrier()

        lanes = lax.iota(jnp.int32, 16)
        nbase = cid * N_NODES
        tbase = cid * TP

        def compute(b, two_tables):
            rowb, erb, er2b, ssb, ctb = (rowv[b], erv[b], erv2, ssv[b],
                                         contribv[b])

            def grp(g, _):
                rb = g * 16
                ridx = rb + lanes
                zc = jnp.zeros((16,), jnp.int32)
                cc = jnp.full((16,), HC, jnp.int32)
                ss16 = plsc.load_gather(ssb, [ridx, zc])
                sd16 = plsc.load_gather(rowb, [ridx, cc])
                se16 = plsc.load_gather(erb, [ridx, cc])
                if two_tables:
                    se16 = se16 + plsc.load_gather(er2b, [ridx, cc])
                p = ss16 + sd16 + se16
                p = jnp.where(p > 0, p, ALPHA * p)
                wv = jnp.exp(-p)

                def edge(j, _):
                    e = rb + j
                    wj = jnp.take_along_axis(
                        wv, jnp.full((16,), j, jnp.int32), axis=0,
                        mode="promise_in_bounds")
                    for k in range(HC // 16):
                        f = k * 16
                        v = rowb[e, pl.ds(f, 16)] + erb[e, pl.ds(f, 16)]
                        if two_tables:
                            v = v + er2b[e, pl.ds(f, 16)]
                        ctb[e, pl.ds(f, 16)] = v * wj
                    ctb[e, pl.ds(HC, 16)] = jnp.where(lanes == 0, wj, 0.0)
                    return 0
                lax.fori_loop(0, 16, edge, 0, unroll=8)
                return 0
            lax.fori_loop(0, C // 16, grp, 0)

        def wait_idx(b):
            pltpu.make_async_copy(src1.at[pl.ds(0, C)], srcv[b],
                                  semidx[b]).wait()

        def wait_row(b, sem):
            pltpu.make_async_copy(dstrow.at[pl.ds(0, C)], rowv[b], sem).wait()

        def drain_scatter(b):
            pltpu.make_async_copy(dstrow.at[pl.ds(0, C)], contribv[b],
                                  semsc[b]).wait()

        def finish(b, two_tables):
            wait_row(b, semr[b])
            pltpu.make_async_copy(sstab.at[pl.ds(0, C)], ssv[b],
                                  semss[b]).wait()
            wait_row(b, seme[b])
            if two_tables:
                pltpu.async_copy(table.at[t2v[b]], erv2, seme2).wait()
            compute(b, two_tables)
            pltpu.async_copy(contribv[b], accs.at[srcs[b]], semsc[b],
                             add=True)

        def make_phase(ch, ew, srca, dsta, n_idx, two_tables, a_tables,
                       seq_base):
            def issue_idx(i, b):
                base = sid * ew + i * C
                pltpu.async_copy(srca.at[pl.ds(base, C)], srcv[b], semidx[b])
                pltpu.async_copy(dsta.at[pl.ds(base, C)], dstv[b], semidx[b])
                if a_tables >= 1:
                    ta = era if a_tables == 1 and not two_tables else t0
                    pltpu.async_copy(ta.at[pl.ds(base, C)], tv[b], semidx[b])
                if a_tables == 2:
                    pltpu.async_copy(t1.at[pl.ds(base, C)], t2v[b], semidx[b])

            def launch(i, b):
                for _ in range(n_idx):
                    wait_idx(b)

                def bias_grp(g, _):
                    sl = pl.ds(g * 16, 16)
                    s = srcv[b][sl]
                    srcs[b][sl] = s
                    srcb[b][sl] = s + nbase
                    dstv[b][sl] = dstv[b][sl] + nbase
                    if a_tables >= 1:
                        tv[b][sl] = tv[b][sl] + tbase
                    if a_tables == 2:
                        t2v[b][sl] = t2v[b][sl] + tbase
                    return 0
                lax.fori_loop(0, C // 16, bias_grp, 0)
                pltpu.async_copy(dstrow.at[dstv[b]], rowv[b], semr[b])
                pltpu.async_copy(sstab.at[srcb[b]], ssv[b], semss[b])
                if seq_base is not None:
                    base = seq_base + sid * ew + i * C
                    pltpu.async_copy(era.at[pl.ds(base, C)], erv[b], seme[b])
                else:
                    pltpu.async_copy(table.at[tv[b]], erv[b], seme[b])

            def run():
                issue_idx(0, 0)
                issue_idx(1, 1)
                launch(0, 0)

                def step(i2, _):
                    for b in (0, 1):
                        i = i2 * 2 + b
                        if b == 1:
                            drain_scatter(0)
                        else:
                            @pl.when(i2 >= 1)
                            def _():
                                drain_scatter(1)
                        if b == 0:
                            launch(i + 1, 1)
                        else:
                            @pl.when(i2 < ch // 2 - 1)
                            def _():
                                launch(i + 1, 0)
                        finish(b, two_tables)

                        @pl.when(i2 < ch // 2 - 1)
                        def _():
                            issue_idx(i + 2, b)
                    return 0
                lax.fori_loop(0, ch // 2, step, 0)
                drain_scatter(1)
            return run

        if seq_er:
            make_phase(ch1, ew1, src1, dst1, 2, False, 0, cid * E1P)()
        else:
            make_phase(ch1, ew1, src1, dst1, 3, False, 1, None)()
        make_phase(ch2, ew2, src2, dst2, 4, True, 2, None)()

        plsc.subcore_barrier()
        pltpu.sync_copy(accs.at[pl.ds(sid * RPS, RPS)],
                        acc_out.at[cid, pl.ds(sid * RPS, RPS)])

    idx_t = pltpu.VMEM((C,), jnp.int32)
    row_t = pltpu.VMEM((C, HROW), jnp.float32)
    return pl.kernel(
        body,
        out_type=jax.ShapeDtypeStruct((2, NP, HROW), jnp.float32),
        mesh=mesh,
        compiler_params=pltpu.CompilerParams(needs_layout_passes=False,
                                             use_tc_tiling_on_sc=False),
        scratch_types=(
            [idx_t] * 12
            + [row_t, row_t]
            + [pltpu.VMEM((C, SSW), jnp.float32)] * 2
            + [row_t, row_t]
            + [row_t, row_t]
            + [row_t]
            + [pltpu.VMEM_SHARED((NP, HROW), jnp.float32)]
            + [pltpu.SemaphoreType.DMA] * 11
        ),
    )


def _halves(payload, scal):
    m = payload.shape[0]
    z = jnp.zeros((m, HROW - HC - 1), payload.dtype)
    halves = [jnp.concatenate([payload[:, c * HC:(c + 1) * HC],
                               scal[:, c:c + 1], z], axis=1)
              for c in range(2)]
    return jnp.concatenate(halves, axis=0)


def kernel(Corpus_, entity_embeddings, relation_embed, edge_list, edge_type,
           edge_embed, edge_list_nhop, edge_type_nhop, a_heads, a2_heads,
           W, a_out, a2_out):
    f32 = jnp.float32
    x = entity_embeddings.astype(f32)
    rel = relation_embed.astype(f32)
    el = edge_list.astype(jnp.int32)
    et = edge_type.astype(jnp.int32)
    ee = edge_embed.astype(f32)
    eln = edge_list_nhop.astype(jnp.int32)
    etn = edge_type_nhop.astype(jnp.int32)
    a_heads = a_heads.astype(f32)
    a2_heads = a2_heads.astype(f32)

    a_s = a_heads[:, :, :NFEAT]
    a_d = a_heads[:, :, NFEAT:2 * NFEAT]
    a_r = a_heads[:, :, 2 * NFEAT:]
    a2s = jnp.einsum("hij,hjk->hk", a2_heads, a_s)
    a2d = jnp.einsum("hij,hjk->hk", a2_heads, a_d)
    a2r = jnp.einsum("hij,hjk->hk", a2_heads, a_r)

    as2 = a_out[:, :128].astype(f32)
    ad2 = a_out[:, 128:256].astype(f32)
    ar2 = a_out[:, 256:].astype(f32)
    a2o = a2_out.astype(f32)

    b1 = jnp.concatenate(
        [a_s[0].T, a_s[1].T, a_d[0].T, a_d[1].T,
         a2s.T, a2d.T, jnp.zeros((128, 124), f32)], axis=1)
    p1 = _mm(x, b1, bm=400)
    xs1 = p1[:, :128]
    dstrow1 = _halves(p1[:, 128:256], p1[:, 258:260])
    sstab1 = jnp.concatenate(
        [jnp.stack([p1[:, 256], p1[:, 257]])[:, :, None],
         jnp.zeros((2, N_NODES, SSW - 1), f32)],
        axis=2).reshape(2 * N_NODES, SSW)

    zc = jnp.zeros((128, HROW - HC - 1), f32)
    b2h = jnp.stack([jnp.concatenate([a_r[c].T, a2r[c:c + 1].T, zc], axis=1)
                     for c in range(2)])
    er1 = _er_tables(ee, b2h)

    b2 = jnp.concatenate(
        [a_r[0].T, a_r[1].T, a2r.T, jnp.zeros((128, 14), f32)], axis=1)
    tr1 = _mm(rel, b2, bm=256)
    dead1 = jnp.zeros((1, 130), f32).at[0, 128:130].set(1e30)
    tr1 = jnp.concatenate(
        [tr1[:, :130], dead1, jnp.zeros((TP - N_REL - 1, 130), f32)], axis=0)
    table1 = _halves(tr1[:, :128], tr1[:, 128:130])

    out_rel = _mm(rel, W.astype(f32), bm=256)

    b5 = jnp.concatenate(
        [ar2.T, (a2o @ ar2).T, jnp.zeros((128, 15), f32)], axis=1)
    t2 = _mm(out_rel, b5, bm=256)
    dead2 = jnp.zeros((1, 129), f32).at[0, 128:129].set(1e30)
    t2 = jnp.concatenate(
        [t2[:, :129], dead2, jnp.zeros((TP - N_REL - 1, 129), f32)], axis=0)
    table2 = _halves(t2[:, :128], t2[:, 128:129][:, [0, 0]])

    src1 = jnp.pad(el[0], (0, E1P - E1))
    dst1 = jnp.pad(el[1], (0, E1P - E1))
    ty1 = jnp.pad(et, (0, E1P - E1), constant_values=N_REL)
    src2 = jnp.pad(eln[0], (0, E2P - E2))
    dst2 = jnp.pad(eln[1], (0, E2P - E2))
    t0 = jnp.pad(etn[:, 0], (0, E2P - E2), constant_values=N_REL)
    t1 = jnp.pad(etn[:, 1], (0, E2P - E2), constant_values=N_REL)

    acc1 = _make_edge_kernel(True)(src1, dst1, er1, src2, dst2, t0, t1,
                                   table1, dstrow1, sstab1)

    b4 = jnp.concatenate(
        [as2.T, ad2.T, (a2o @ as2).T, (a2o @ ad2).T,
         jnp.zeros((128, 126), f32)], axis=1)
    p2 = _combine1(acc1, xs1, b4)
    xs2 = p2[:, :128]
    dstrow2 = _halves(p2[:, 128:256], p2[:, 257:258][:, [0, 0]])
    sstab2 = jnp.concatenate(
        [jnp.stack([p2[:, 256], p2[:, 256]])[:, :, None],
         jnp.zeros((2, N_NODES, SSW - 1), f32)],
        axis=2).reshape(2 * N_NODES, SSW)

    acc2 = _make_edge_kernel(False)(src1, dst1, ty1, src2, dst2, t0, t1,
                                    table2, dstrow2, sstab2)

    out = _combine2(acc2, xs2)
    return (out, out_rel)
